# Initial kernel scaffold; baseline (speedup 1.0000x reference)
import jax
import jax.numpy as jnp
from jax.experimental import pallas as pl


def kernel(z, edge_index):
    raise NotImplementedError("write your pallas kernel here")


# speedup vs baseline: 4.0930x; 4.0930x over previous
import functools

import jax
import jax.numpy as jnp
from jax import lax
from jax.experimental import pallas as pl
from jax.experimental.pallas import tpu as pltpu
from jax.experimental.pallas import tpu_sc as plsc

_D = 128
_L = 16
_NW = 32
_C = 80


@functools.partial(jax.jit, static_argnums=(3,))
def _decode(z, src, dst, n_edges):
    per_w = n_edges // _NW
    n_chunks = per_w // _C

    mesh = plsc.VectorSubcoreMesh(core_axis_name="c", subcore_axis_name="s")

    @functools.partial(
        pl.kernel,
        mesh=mesh,
        out_type=jax.ShapeDtypeStruct((n_edges,), jnp.float32),
        scratch_types=[
            pltpu.VMEM((_C,), jnp.int32),
            pltpu.VMEM((_C,), jnp.int32),
            pltpu.VMEM((_C, _D), jnp.float32),
            pltpu.VMEM((_C, _D), jnp.float32),
            pltpu.VMEM((per_w,), jnp.float32),
            pltpu.SemaphoreType.DMA,
            pltpu.SemaphoreType.DMA,
        ],
        compiler_params=pltpu.CompilerParams(needs_layout_passes=False),
    )
    def body(z_hbm, src_hbm, dst_hbm, out_hbm,
             sidx_v, didx_v, srows_v, drows_v, out_v, sem_s, sem_d):
        wid = lax.axis_index("s") * 2 + lax.axis_index("c")
        base = wid * per_w
        lane = lax.iota(jnp.int32, _L)

        def chunk_body(i, _):
            off = base + i * _C
            pltpu.sync_copy(src_hbm.at[pl.ds(off, _C)], sidx_v)
            pltpu.sync_copy(dst_hbm.at[pl.ds(off, _C)], didx_v)
            cps = pltpu.async_copy(z_hbm.at[sidx_v], srows_v, sem_s)
            cpd = pltpu.async_copy(z_hbm.at[didx_v], drows_v, sem_d)
            cps.wait()
            cpd.wait()

            def group_body(g, _):
                def edge_body(k, res):
                    e = g * _L + k
                    acc = jnp.zeros((_L,), jnp.float32)
                    for j in range(_D // _L):
                        s = srows_v[e, pl.ds(j * _L, _L)]
                        d = drows_v[e, pl.ds(j * _L, _L)]
                        acc = acc + s * d
                    tot = jnp.sum(acc)
                    return jnp.where(lane == k, tot, res)

                res = lax.fori_loop(
                    0, _L, edge_body, jnp.zeros((_L,), jnp.float32))
                out_v[pl.ds(i * _C + g * _L, _L)] = res
                return 0

            lax.fori_loop(0, _C // _L, group_body, 0)
            return 0

        lax.fori_loop(0, n_chunks, chunk_body, 0)
        pltpu.sync_copy(out_v, out_hbm.at[pl.ds(base, per_w)])

    return body(z, src, dst)


def kernel(z, edge_index):
    src = edge_index[0].astype(jnp.int32)
    dst = edge_index[1].astype(jnp.int32)
    return _decode(z, src, dst, edge_index.shape[1])
